# baseline (speedup 1.0000x reference)
import dataclasses
import functools

import jax
import jax.numpy as jnp
from jax import lax
from jax.experimental import pallas as pl
from jax.experimental.pallas import tpu as pltpu
from jax.experimental.pallas import tpu_sc as plsc

_PERCENTILE = 90
_EPS = 1e-6
_SIGN = -2147483648
_N_WORKERS = 32
_SC_ROWS = 512


def _sc_threshold_body(x_hbm, thr_hbm, xrow, cand, hist, thrbuf, *, n, k,
                       rows_per_w):
    nit = n // 64
    c = lax.axis_index("c")
    s = lax.axis_index("s")
    wid = s * 2 + c
    sign = jnp.int32(_SIGN)
    lane = lax.iota(jnp.int32, 16)
    ones = jnp.ones((16,), jnp.int32)
    zeros16 = jnp.zeros((16,), jnp.int32)
    pad16 = jnp.full((16,), sign, jnp.int32)
    kk = jnp.int32(k)

    def zero_hist():
        @pl.loop(0, 256)
        def _z(b):
            for p in range(4):
                hist[pl.ds(p * 4096 + b * 16, 16)] = zeros16

    def fold_and_scan(base):
        @pl.loop(0, 256)
        def _f(b):
            h = (hist[pl.ds(b * 16, 16)]
                 + hist[pl.ds(4096 + b * 16, 16)]) + \
                (hist[pl.ds(2 * 4096 + b * 16, 16)]
                 + hist[pl.ds(3 * 4096 + b * 16, 16)])
            hist[pl.ds(b * 16, 16)] = h

        def scan_body(t, carry):
            acc, b0, cab = carry
            g = jnp.int32(255) - t
            gs = jnp.sum(hist[pl.ds(g * 16, 16)])
            acc2 = acc + gs
            hit = jnp.logical_and(base + acc2 >= kk, base + acc < kk)
            b0 = jnp.where(hit, g, b0)
            cab = jnp.where(hit, acc, cab)
            return acc2, b0, cab

        _, b0, cab = lax.fori_loop(
            0, 256, scan_body,
            (jnp.int32(0), jnp.int32(0), jnp.int32(0)))
        return b0, cab

    @pl.loop(0, rows_per_w)
    def _row(r):
        row = wid * rows_per_w + r
        pltpu.sync_copy(x_hbm.at[row], xrow)

        zero_hist()

        @pl.loop(0, nit)
        def _h(j):
            for p in range(4):
                v = xrow[pl.ds((j * 4 + p) * 16, 16)]
                i = plsc.bitcast(v, jnp.int32)
                flip = lax.shift_right_arithmetic(i, 31)
                keyu = i ^ (flip | sign)
                bin_ = lax.shift_right_logical(keyu, 24)
                plsc.addupdate_scatter(
                    hist, [p * 4096 + bin_ * 16 + lane], ones)

        b0, cabove = fold_and_scan(jnp.int32(0))

        def comp_body(j, off):
            for p in range(4):
                v = xrow[pl.ds((j * 4 + p) * 16, 16)]
                i = plsc.bitcast(v, jnp.int32)
                flip = lax.shift_right_arithmetic(i, 31)
                keyu = i ^ (flip | sign)
                keys = keyu ^ sign
                m = lax.shift_right_logical(keyu, 24) == b0
                mi = m.astype(jnp.int32)
                pos = (plsc.cumsum(mi) - 1) + off
                plsc.store_scatter(cand, [pos], keys, mask=m)
                pc = plsc.all_reduce_population_count(m)
                off = off + pc[0]
            return off

        off1 = lax.fori_loop(0, nit, comp_body, jnp.int32(0))
        for p in range(4):
            cand[pl.ds(off1 + p * 16, 16)] = pad16

        zero_hist()
        nit2 = lax.shift_right_logical(off1 + 63, 6)

        def h2_body(j, dummy):
            for p in range(4):
                base = (j * 4 + p) * 16
                v = cand[pl.ds(base, 16)]
                valid = (base + lane) < off1
                bin_ = lax.shift_right_logical(v, 16) & 255
                plsc.addupdate_scatter(
                    hist, [p * 4096 + bin_ * 16 + lane], ones, mask=valid)
            return dummy

        lax.fori_loop(0, nit2, h2_body, jnp.int32(0))
        b1, c2 = fold_and_scan(cabove)
        cabove2 = cabove + c2

        def comp2_body(j, off):
            for p in range(4):
                base = (j * 4 + p) * 16
                v = cand[pl.ds(base, 16)]
                valid = (base + lane) < off1
                m = jnp.logical_and(
                    (lax.shift_right_logical(v, 16) & 255) == b1, valid)
                mi = m.astype(jnp.int32)
                pos = (plsc.cumsum(mi) - 1) + off
                plsc.store_scatter(cand, [pos], v, mask=m)
                pc = plsc.all_reduce_population_count(m)
                off = off + pc[0]
            return off

        off2 = lax.fori_loop(0, nit2, comp2_body, jnp.int32(0))
        cand[pl.ds(off2, 16)] = pad16
        nvd = lax.shift_right_logical(off2 + 15, 4)

        def bit_body(t, prefix):
            bit = jnp.int32(1) << (jnp.int32(15) - t)
            candthr = prefix | bit

            def cnt_body(j, cnt):
                v = cand[pl.ds(j * 16, 16)]
                pc = plsc.all_reduce_population_count(v >= candthr)
                return cnt + pc[0]

            cnt = lax.fori_loop(0, nvd, cnt_body, jnp.int32(0))
            return jnp.where(cabove2 + cnt >= kk, candthr, prefix)

        key_s = lax.fori_loop(
            0, 16, bit_body,
            (jnp.left_shift(b0, 24) | jnp.left_shift(b1, 16)) ^ sign)
        kv = jnp.full((16,), key_s, jnp.int32)
        iv = jnp.where(kv >= 0, kv, kv ^ jnp.int32(0x7FFFFFFF))
        fv = plsc.bitcast(iv, jnp.float32)
        rv = jnp.full((16,), r, jnp.int32)
        plsc.store_scatter(thrbuf, [rv], fv, mask=lane == 0)

    pltpu.sync_copy(thrbuf, thr_hbm.at[wid])


def _sc_thresholds(x2, n, k, m):
    rows_per_w = m // _N_WORKERS
    mesh = plsc.VectorSubcoreMesh(core_axis_name="c", subcore_axis_name="s")
    body = functools.partial(_sc_threshold_body, n=n, k=k,
                             rows_per_w=rows_per_w)
    cp = pltpu.CompilerParams()
    if "needs_layout_passes" in pltpu.CompilerParams.__dataclass_fields__:
        cp = dataclasses.replace(cp, needs_layout_passes=False)
    kern = pl.kernel(
        body,
        out_type=jax.ShapeDtypeStruct((_N_WORKERS, 16), jnp.float32),
        mesh=mesh,
        compiler_params=cp,
        scratch_types=[
            pltpu.VMEM((n,), jnp.float32),
            pltpu.VMEM((n + 64,), jnp.int32),
            pltpu.VMEM((4 * 4096,), jnp.int32),
            pltpu.VMEM((16,), jnp.float32),
        ],
    )
    thr2 = kern(x2[:m])
    return thr2[:, :rows_per_w].reshape(m, 1)


def _ash_fused_body(x_ref, o_ref, *, k):
    sign = jnp.int32(_SIGN)
    xb = x_ref[...]
    i = jax.lax.bitcast_convert_type(xb, jnp.int32)
    flip = jax.lax.shift_right_arithmetic(i, 31)
    keys = i ^ (flip & jnp.int32(0x7FFFFFFF))
    rows = xb.shape[0]

    n = xb.shape[1]
    nq = n // 8

    def step(b, prefix):
        bit = jnp.int32(1) << (jnp.int32(31) - b)
        cand = prefix | bit
        cand_cmp = cand ^ sign
        m = (keys >= cand_cmp).astype(jnp.int32)
        parts = [jnp.sum(m[:, j * nq:(j + 1) * nq], axis=1, keepdims=True)
                 for j in range(8)]
        while len(parts) > 1:
            parts = [parts[i] + parts[i + 1] for i in range(0, len(parts), 2)]
        cnt = parts[0]
        return jnp.where(cnt >= k, cand, prefix)

    prefix = jax.lax.fori_loop(
        0, 32, step, jnp.zeros((rows, 1), jnp.int32), unroll=4)
    thr = prefix ^ sign
    mask = (keys >= thr).astype(jnp.float32)
    xp = xb * mask
    s_orig = jnp.sum(xb, axis=1, keepdims=True)
    s_prun = jnp.sum(xp, axis=1, keepdims=True)
    o_ref[...] = xp * (s_orig / (s_prun + _EPS))


def _mask_body(x_ref, t_ref, o_ref):
    xb = x_ref[...]
    thr = t_ref[...]
    mask = (xb >= thr).astype(jnp.float32)
    xp = xb * mask
    s_orig = jnp.sum(xb, axis=1, keepdims=True)
    s_prun = jnp.sum(xp, axis=1, keepdims=True)
    o_ref[...] = xp * (s_orig / (s_prun + _EPS))


@jax.jit
def kernel(x):
    B, C, H, W = x.shape
    n = H * W
    k = int(n * (1.0 - _PERCENTILE / 100.0))
    rows_total = B * C
    x2 = x.reshape(rows_total, n)
    m = _SC_ROWS
    R = 32

    out_hi = pl.pallas_call(
        functools.partial(_ash_fused_body, k=k),
        grid=((rows_total - m) // R,),
        in_specs=[pl.BlockSpec((R, n), lambda i: (i, 0))],
        out_specs=pl.BlockSpec((R, n), lambda i: (i, 0)),
        out_shape=jax.ShapeDtypeStruct((rows_total - m, n), jnp.float32),
        compiler_params=pltpu.CompilerParams(
            dimension_semantics=("parallel",)),
    )(x2[m:])

    thr_sc = _sc_thresholds(x2, n, k, m)

    out_lo = pl.pallas_call(
        _mask_body,
        grid=(m // R,),
        in_specs=[pl.BlockSpec((R, n), lambda i: (i, 0)),
                  pl.BlockSpec((R, 1), lambda i: (i, 0))],
        out_specs=pl.BlockSpec((R, n), lambda i: (i, 0)),
        out_shape=jax.ShapeDtypeStruct((m, n), jnp.float32),
        compiler_params=pltpu.CompilerParams(
            dimension_semantics=("parallel",)),
    )(x2[:m], thr_sc)

    out = jnp.concatenate([out_lo, out_hi], axis=0)
    return out.reshape(B, C, H, W)


# speedup vs baseline: 1.6969x; 1.6969x over previous
import dataclasses
import functools

import jax
import jax.numpy as jnp
from jax import lax
from jax.experimental import pallas as pl
from jax.experimental.pallas import tpu as pltpu
from jax.experimental.pallas import tpu_sc as plsc

_PERCENTILE = 90
_EPS = 1e-6
_SIGN = -2147483648
_N_WORKERS = 32
_SC_ROWS = 256


def _sc_threshold_body(x_hbm, thr_hbm, xrow, cand, hist, thrbuf, *, n, k,
                       rows_per_w):
    nit = n // 64
    c = lax.axis_index("c")
    s = lax.axis_index("s")
    wid = s * 2 + c
    sign = jnp.int32(_SIGN)
    lane = lax.iota(jnp.int32, 16)
    ones = jnp.ones((16,), jnp.int32)
    zeros16 = jnp.zeros((16,), jnp.int32)
    pad16 = jnp.full((16,), sign, jnp.int32)
    kk = jnp.int32(k)

    def zero_hist():
        @pl.loop(0, 256)
        def _z(b):
            for p in range(4):
                hist[pl.ds(p * 4096 + b * 16, 16)] = zeros16

    def fold_and_scan(base):
        def scan_body(t, carry):
            acc, b0, cab = carry
            g = jnp.int32(255) - t
            h = (hist[pl.ds(g * 16, 16)]
                 + hist[pl.ds(4096 + g * 16, 16)]) + \
                (hist[pl.ds(2 * 4096 + g * 16, 16)]
                 + hist[pl.ds(3 * 4096 + g * 16, 16)])
            gs = jnp.sum(h)
            acc2 = acc + gs
            hit = jnp.logical_and(base + acc2 >= kk, base + acc < kk)
            b0 = jnp.where(hit, g, b0)
            cab = jnp.where(hit, acc, cab)
            return acc2, b0, cab

        _, b0, cab = lax.fori_loop(
            0, 256, scan_body,
            (jnp.int32(0), jnp.int32(0), jnp.int32(0)))
        return b0, cab

    @pl.loop(0, rows_per_w)
    def _row(r):
        row = wid * rows_per_w + r
        pltpu.sync_copy(x_hbm.at[row], xrow)

        zero_hist()

        @pl.loop(0, nit)
        def _h(j):
            for p in range(4):
                v = xrow[pl.ds((j * 4 + p) * 16, 16)]
                i = plsc.bitcast(v, jnp.int32)
                flip = lax.shift_right_arithmetic(i, 31)
                keyu = i ^ (flip | sign)
                bin_ = lax.shift_right_logical(keyu, 24)
                plsc.addupdate_scatter(
                    hist, [p * 4096 + bin_ * 16 + lane], ones)

        b0, cabove = fold_and_scan(jnp.int32(0))

        def comp_body(j, off):
            for p in range(4):
                v = xrow[pl.ds((j * 4 + p) * 16, 16)]
                i = plsc.bitcast(v, jnp.int32)
                flip = lax.shift_right_arithmetic(i, 31)
                keyu = i ^ (flip | sign)
                keys = keyu ^ sign
                m = lax.shift_right_logical(keyu, 24) == b0
                mi = m.astype(jnp.int32)
                pos = (plsc.cumsum(mi) - 1) + off
                plsc.store_scatter(cand, [pos], keys, mask=m)
                pc = plsc.all_reduce_population_count(m)
                off = off + pc[0]
            return off

        off1 = lax.fori_loop(0, nit, comp_body, jnp.int32(0))
        for p in range(4):
            cand[pl.ds(off1 + p * 16, 16)] = pad16

        zero_hist()
        nit2 = lax.shift_right_logical(off1 + 63, 6)

        def h2_body(j, dummy):
            for p in range(4):
                base = (j * 4 + p) * 16
                v = cand[pl.ds(base, 16)]
                valid = (base + lane) < off1
                bin_ = lax.shift_right_logical(v, 16) & 255
                plsc.addupdate_scatter(
                    hist, [p * 4096 + bin_ * 16 + lane], ones, mask=valid)
            return dummy

        lax.fori_loop(0, nit2, h2_body, jnp.int32(0))
        b1, c2 = fold_and_scan(cabove)
        cabove2 = cabove + c2

        def comp2_body(j, off):
            for p in range(4):
                base = (j * 4 + p) * 16
                v = cand[pl.ds(base, 16)]
                valid = (base + lane) < off1
                m = jnp.logical_and(
                    (lax.shift_right_logical(v, 16) & 255) == b1, valid)
                mi = m.astype(jnp.int32)
                pos = (plsc.cumsum(mi) - 1) + off
                plsc.store_scatter(cand, [pos], v, mask=m)
                pc = plsc.all_reduce_population_count(m)
                off = off + pc[0]
            return off

        off2 = lax.fori_loop(0, nit2, comp2_body, jnp.int32(0))
        cand[pl.ds(off2, 16)] = pad16
        nvd = lax.shift_right_logical(off2 + 15, 4)

        def bit_body(t, prefix):
            bit = jnp.int32(1) << (jnp.int32(15) - t)
            candthr = prefix | bit

            def cnt_body(j, cnt):
                v = cand[pl.ds(j * 16, 16)]
                pc = plsc.all_reduce_population_count(v >= candthr)
                return cnt + pc[0]

            cnt = lax.fori_loop(0, nvd, cnt_body, jnp.int32(0))
            return jnp.where(cabove2 + cnt >= kk, candthr, prefix)

        key_s = lax.fori_loop(
            0, 16, bit_body,
            (jnp.left_shift(b0, 24) | jnp.left_shift(b1, 16)) ^ sign)
        kv = jnp.full((16,), key_s, jnp.int32)
        iv = jnp.where(kv >= 0, kv, kv ^ jnp.int32(0x7FFFFFFF))
        fv = plsc.bitcast(iv, jnp.float32)
        rv = jnp.full((16,), r, jnp.int32)
        plsc.store_scatter(thrbuf, [rv], fv, mask=lane == 0)

    pltpu.sync_copy(thrbuf, thr_hbm.at[wid])


def _sc_thresholds(x2, n, k, m):
    rows_per_w = m // _N_WORKERS
    mesh = plsc.VectorSubcoreMesh(core_axis_name="c", subcore_axis_name="s")
    body = functools.partial(_sc_threshold_body, n=n, k=k,
                             rows_per_w=rows_per_w)
    cp = pltpu.CompilerParams()
    if "needs_layout_passes" in pltpu.CompilerParams.__dataclass_fields__:
        cp = dataclasses.replace(cp, needs_layout_passes=False)
    kern = pl.kernel(
        body,
        out_type=jax.ShapeDtypeStruct((_N_WORKERS, 16), jnp.float32),
        mesh=mesh,
        compiler_params=cp,
        scratch_types=[
            pltpu.VMEM((n,), jnp.float32),
            pltpu.VMEM((n + 64,), jnp.int32),
            pltpu.VMEM((4 * 4096,), jnp.int32),
            pltpu.VMEM((16,), jnp.float32),
        ],
    )
    thr2 = kern(x2[:m])
    return thr2[:, :rows_per_w].reshape(m, 1)


def _ash_fused_body(x_ref, o_ref, *, k):
    sign = jnp.int32(_SIGN)
    xb = x_ref[...]
    i = jax.lax.bitcast_convert_type(xb, jnp.int32)
    flip = jax.lax.shift_right_arithmetic(i, 31)
    keys = i ^ (flip & jnp.int32(0x7FFFFFFF))
    rows = xb.shape[0]

    n = xb.shape[1]
    nq = n // 8

    def step(b, prefix):
        bit = jnp.int32(1) << (jnp.int32(31) - b)
        cand = prefix | bit
        cand_cmp = cand ^ sign
        m = (keys >= cand_cmp).astype(jnp.int32)
        parts = [jnp.sum(m[:, j * nq:(j + 1) * nq], axis=1, keepdims=True)
                 for j in range(8)]
        while len(parts) > 1:
            parts = [parts[i] + parts[i + 1] for i in range(0, len(parts), 2)]
        cnt = parts[0]
        return jnp.where(cnt >= k, cand, prefix)

    prefix = jax.lax.fori_loop(
        0, 32, step, jnp.zeros((rows, 1), jnp.int32), unroll=4)
    thr = prefix ^ sign
    mask = (keys >= thr).astype(jnp.float32)
    xp = xb * mask
    s_orig = jnp.sum(xb, axis=1, keepdims=True)
    s_prun = jnp.sum(xp, axis=1, keepdims=True)
    o_ref[...] = xp * (s_orig / (s_prun + _EPS))


def _mask_body(x_ref, t_ref, o_ref):
    xb = x_ref[...]
    thr = t_ref[...]
    mask = (xb >= thr).astype(jnp.float32)
    xp = xb * mask
    s_orig = jnp.sum(xb, axis=1, keepdims=True)
    s_prun = jnp.sum(xp, axis=1, keepdims=True)
    o_ref[...] = xp * (s_orig / (s_prun + _EPS))


@jax.jit
def kernel(x):
    B, C, H, W = x.shape
    n = H * W
    k = int(n * (1.0 - _PERCENTILE / 100.0))
    rows_total = B * C
    x2 = x.reshape(rows_total, n)
    m = _SC_ROWS
    R = 32

    out_hi = pl.pallas_call(
        functools.partial(_ash_fused_body, k=k),
        grid=((rows_total - m) // R,),
        in_specs=[pl.BlockSpec((R, n), lambda i: (i, 0))],
        out_specs=pl.BlockSpec((R, n), lambda i: (i, 0)),
        out_shape=jax.ShapeDtypeStruct((rows_total - m, n), jnp.float32),
        compiler_params=pltpu.CompilerParams(
            dimension_semantics=("parallel",)),
    )(x2[m:])

    thr_sc = _sc_thresholds(x2, n, k, m)

    out_lo = pl.pallas_call(
        _mask_body,
        grid=(m // R,),
        in_specs=[pl.BlockSpec((R, n), lambda i: (i, 0)),
                  pl.BlockSpec((R, 1), lambda i: (i, 0))],
        out_specs=pl.BlockSpec((R, n), lambda i: (i, 0)),
        out_shape=jax.ShapeDtypeStruct((m, n), jnp.float32),
        compiler_params=pltpu.CompilerParams(
            dimension_semantics=("parallel",)),
    )(x2[:m], thr_sc)

    out = jnp.concatenate([out_lo, out_hi], axis=0)
    return out.reshape(B, C, H, W)


# speedup vs baseline: 1.8915x; 1.1147x over previous
import dataclasses
import functools

import jax
import jax.numpy as jnp
from jax import lax
from jax.experimental import pallas as pl
from jax.experimental.pallas import tpu as pltpu
from jax.experimental.pallas import tpu_sc as plsc

_PERCENTILE = 90
_EPS = 1e-6
_SIGN = -2147483648
_N_WORKERS = 32
_SC_ROWS = 256


def _sc_threshold_body(x_hbm, thr_hbm, xrow, cand, hist, thrbuf, *, n, k,
                       rows_per_w):
    nit = n // 64
    c = lax.axis_index("c")
    s = lax.axis_index("s")
    wid = s * 2 + c
    sign = jnp.int32(_SIGN)
    lane = lax.iota(jnp.int32, 16)
    ones = jnp.ones((16,), jnp.int32)
    zeros16 = jnp.zeros((16,), jnp.int32)
    pad16 = jnp.full((16,), sign, jnp.int32)
    kk = jnp.int32(k)

    def zero_hist():
        @pl.loop(0, 256)
        def _z(b):
            for p in range(4):
                hist[pl.ds(p * 4096 + b * 16, 16)] = zeros16

    def fold_and_scan(base):
        def scan_body(t, carry):
            acc, b0, cab = carry
            g = jnp.int32(255) - t
            h = (hist[pl.ds(g * 16, 16)]
                 + hist[pl.ds(4096 + g * 16, 16)]) + \
                (hist[pl.ds(2 * 4096 + g * 16, 16)]
                 + hist[pl.ds(3 * 4096 + g * 16, 16)])
            gs = jnp.sum(h)
            acc2 = acc + gs
            hit = jnp.logical_and(base + acc2 >= kk, base + acc < kk)
            b0 = jnp.where(hit, g, b0)
            cab = jnp.where(hit, acc, cab)
            return acc2, b0, cab

        _, b0, cab = lax.fori_loop(
            0, 256, scan_body,
            (jnp.int32(0), jnp.int32(0), jnp.int32(0)))
        return b0, cab

    @pl.loop(0, rows_per_w)
    def _row(r):
        row = wid * rows_per_w + r
        pltpu.sync_copy(x_hbm.at[row], xrow)

        zero_hist()

        @pl.loop(0, nit)
        def _h(j):
            for p in range(4):
                v = xrow[pl.ds((j * 4 + p) * 16, 16)]
                i = plsc.bitcast(v, jnp.int32)
                flip = lax.shift_right_arithmetic(i, 31)
                keyu = i ^ (flip | sign)
                bin_ = lax.shift_right_logical(keyu, 24)
                plsc.addupdate_scatter(
                    hist, [p * 4096 + bin_ * 16 + lane], ones)
                cand[pl.ds((j * 4 + p) * 16, 16)] = keyu ^ sign

        b0, cabove = fold_and_scan(jnp.int32(0))

        def comp_body(j, off):
            for p in range(4):
                keys = cand[pl.ds((j * 4 + p) * 16, 16)]
                m = lax.shift_right_logical(keys ^ sign, 24) == b0
                plsc.store_compressed(cand.at[pl.ds(off, 16)], keys, mask=m)
                pc = plsc.all_reduce_population_count(m)
                off = off + pc[0]
            return off

        off1 = lax.fori_loop(0, nit, comp_body, jnp.int32(0))
        for p in range(4):
            cand[pl.ds(off1 + p * 16, 16)] = pad16

        zero_hist()
        nit2 = lax.shift_right_logical(off1 + 63, 6)

        def h2_body(j, dummy):
            for p in range(4):
                base = (j * 4 + p) * 16
                v = cand[pl.ds(base, 16)]
                valid = (base + lane) < off1
                bin_ = lax.shift_right_logical(v, 16) & 255
                plsc.addupdate_scatter(
                    hist, [p * 4096 + bin_ * 16 + lane], ones, mask=valid)
            return dummy

        lax.fori_loop(0, nit2, h2_body, jnp.int32(0))
        b1, c2 = fold_and_scan(cabove)
        cabove2 = cabove + c2

        def comp2_body(j, off):
            for p in range(4):
                base = (j * 4 + p) * 16
                v = cand[pl.ds(base, 16)]
                valid = (base + lane) < off1
                m = jnp.logical_and(
                    (lax.shift_right_logical(v, 16) & 255) == b1, valid)
                plsc.store_compressed(cand.at[pl.ds(off, 16)], v, mask=m)
                pc = plsc.all_reduce_population_count(m)
                off = off + pc[0]
            return off

        off2 = lax.fori_loop(0, nit2, comp2_body, jnp.int32(0))
        cand[pl.ds(off2, 16)] = pad16
        nvd = lax.shift_right_logical(off2 + 15, 4)

        def bit_body(t, prefix):
            bit = jnp.int32(1) << (jnp.int32(15) - t)
            candthr = prefix | bit

            def cnt_body(j, cnt):
                v = cand[pl.ds(j * 16, 16)]
                pc = plsc.all_reduce_population_count(v >= candthr)
                return cnt + pc[0]

            cnt = lax.fori_loop(0, nvd, cnt_body, jnp.int32(0))
            return jnp.where(cabove2 + cnt >= kk, candthr, prefix)

        key_s = lax.fori_loop(
            0, 16, bit_body,
            (jnp.left_shift(b0, 24) | jnp.left_shift(b1, 16)) ^ sign)
        kv = jnp.full((16,), key_s, jnp.int32)
        iv = jnp.where(kv >= 0, kv, kv ^ jnp.int32(0x7FFFFFFF))
        fv = plsc.bitcast(iv, jnp.float32)
        rv = jnp.full((16,), r, jnp.int32)
        plsc.store_scatter(thrbuf, [rv], fv, mask=lane == 0)

    pltpu.sync_copy(thrbuf, thr_hbm.at[wid])


def _sc_thresholds(x2, n, k, m):
    rows_per_w = m // _N_WORKERS
    mesh = plsc.VectorSubcoreMesh(core_axis_name="c", subcore_axis_name="s")
    body = functools.partial(_sc_threshold_body, n=n, k=k,
                             rows_per_w=rows_per_w)
    cp = pltpu.CompilerParams()
    if "needs_layout_passes" in pltpu.CompilerParams.__dataclass_fields__:
        cp = dataclasses.replace(cp, needs_layout_passes=False)
    kern = pl.kernel(
        body,
        out_type=jax.ShapeDtypeStruct((_N_WORKERS, 16), jnp.float32),
        mesh=mesh,
        compiler_params=cp,
        scratch_types=[
            pltpu.VMEM((n,), jnp.float32),
            pltpu.VMEM((n + 64,), jnp.int32),
            pltpu.VMEM((4 * 4096,), jnp.int32),
            pltpu.VMEM((16,), jnp.float32),
        ],
    )
    thr2 = kern(x2[:m])
    return thr2[:, :rows_per_w].reshape(m, 1)


def _ash_fused_body(x_ref, o_ref, *, k):
    sign = jnp.int32(_SIGN)
    xb = x_ref[...]
    i = jax.lax.bitcast_convert_type(xb, jnp.int32)
    flip = jax.lax.shift_right_arithmetic(i, 31)
    keys = i ^ (flip & jnp.int32(0x7FFFFFFF))
    rows = xb.shape[0]

    n = xb.shape[1]
    nq = n // 8

    def step(b, prefix):
        bit = jnp.int32(1) << (jnp.int32(31) - b)
        cand = prefix | bit
        cand_cmp = cand ^ sign
        m = (keys >= cand_cmp).astype(jnp.int32)
        parts = [jnp.sum(m[:, j * nq:(j + 1) * nq], axis=1, keepdims=True)
                 for j in range(8)]
        while len(parts) > 1:
            parts = [parts[i] + parts[i + 1] for i in range(0, len(parts), 2)]
        cnt = parts[0]
        return jnp.where(cnt >= k, cand, prefix)

    prefix = jax.lax.fori_loop(
        0, 32, step, jnp.zeros((rows, 1), jnp.int32), unroll=4)
    thr = prefix ^ sign
    mask = (keys >= thr).astype(jnp.float32)
    xp = xb * mask
    s_orig = jnp.sum(xb, axis=1, keepdims=True)
    s_prun = jnp.sum(xp, axis=1, keepdims=True)
    o_ref[...] = xp * (s_orig / (s_prun + _EPS))


def _mask_body(x_ref, t_ref, o_ref):
    xb = x_ref[...]
    thr = t_ref[...]
    mask = (xb >= thr).astype(jnp.float32)
    xp = xb * mask
    s_orig = jnp.sum(xb, axis=1, keepdims=True)
    s_prun = jnp.sum(xp, axis=1, keepdims=True)
    o_ref[...] = xp * (s_orig / (s_prun + _EPS))


@jax.jit
def kernel(x):
    B, C, H, W = x.shape
    n = H * W
    k = int(n * (1.0 - _PERCENTILE / 100.0))
    rows_total = B * C
    x2 = x.reshape(rows_total, n)
    m = _SC_ROWS
    R = 32

    out_hi = pl.pallas_call(
        functools.partial(_ash_fused_body, k=k),
        grid=((rows_total - m) // R,),
        in_specs=[pl.BlockSpec((R, n), lambda i: (i, 0))],
        out_specs=pl.BlockSpec((R, n), lambda i: (i, 0)),
        out_shape=jax.ShapeDtypeStruct((rows_total - m, n), jnp.float32),
        compiler_params=pltpu.CompilerParams(
            dimension_semantics=("parallel",)),
    )(x2[m:])

    thr_sc = _sc_thresholds(x2, n, k, m)

    out_lo = pl.pallas_call(
        _mask_body,
        grid=(m // R,),
        in_specs=[pl.BlockSpec((R, n), lambda i: (i, 0)),
                  pl.BlockSpec((R, 1), lambda i: (i, 0))],
        out_specs=pl.BlockSpec((R, n), lambda i: (i, 0)),
        out_shape=jax.ShapeDtypeStruct((m, n), jnp.float32),
        compiler_params=pltpu.CompilerParams(
            dimension_semantics=("parallel",)),
    )(x2[:m], thr_sc)

    out = jnp.concatenate([out_lo, out_hi], axis=0)
    return out.reshape(B, C, H, W)
